# baseline (device time: 363492 ns/iter reference)
import jax
import jax.numpy as jnp
import numpy as np
from jax import lax
from jax.experimental import pallas as pl
from jax.experimental.pallas import tpu as pltpu

N_DEV = 4
SQ = 1024
S = N_DEV * SQ
D = 1024
H = 8
DH = 128
QB = 256
NQB = SQ // QB
SCALE = 0.08838834764831843
SCALE2 = SCALE * 1.4426950408889634

F32 = jnp.float32
BF16 = jnp.bfloat16

_DN = lambda lc, rc: (((lc,), (rc,)), ((), ()))


def _rope_tables():
    inv = 1.0 / (10000.0 ** (np.arange(0, DH, 2) / DH))
    pos = np.arange(S)[:, None] * inv[None, :]
    cos = np.repeat(np.cos(pos), 2, axis=-1)
    sin = np.repeat(np.sin(pos), 2, axis=-1)
    R = np.zeros((DH, DH), np.float32)
    for k in range(DH // 2):
        R[2 * k + 1, 2 * k] = -1.0
        R[2 * k, 2 * k + 1] = 1.0
    return cos.astype(np.float32), sin.astype(np.float32), R


def _body(x_ref, wq_ref, wk_ref, wv_ref, wo_ref, cosh_ref, sinh_ref, roth_ref,
          out_ref, xg_ref, k_ref, v_ref, qb_ref, po_ref, sb_ref, rs_ref,
          cos_ref, sin_ref, rot_ref,
          ag_send, ag_recv, rs_send, rs_recv, dma_sems, out_sem):
    my = lax.axis_index("i")
    left = lax.rem(my + N_DEV - 1, N_DEV)
    right = lax.rem(my + 1, N_DEV)

    barrier = pltpu.get_barrier_semaphore()
    for nbr in (left, right):
        pl.semaphore_signal(barrier, inc=1, device_id=(nbr,),
                            device_id_type=pl.DeviceIdType.MESH)
    pl.semaphore_wait(barrier, 2)

    copies = [
        pltpu.make_async_copy(x_ref, xg_ref.at[pl.ds(my * SQ, SQ), :],
                              dma_sems.at[0]),
        pltpu.make_async_copy(cosh_ref, cos_ref, dma_sems.at[1]),
        pltpu.make_async_copy(sinh_ref, sin_ref, dma_sems.at[2]),
        pltpu.make_async_copy(roth_ref, rot_ref, dma_sems.at[3]),
    ]
    for cp in copies:
        cp.start()
    for cp in copies:
        cp.wait()

    rot = rot_ref[...]

    def _rope(tf, cos_s, sin_s):
        tr = lax.dot_general(tf.astype(BF16), rot, _DN(1, 0),
                             preferred_element_type=F32)
        return tf * cos_s + tr * sin_s

    def kv_chunk(o):
        rows = pl.ds(o * SQ, SQ)
        xs = xg_ref[rows, :]
        cos_s = cos_ref[rows, :].astype(F32)
        sin_s = sin_ref[rows, :].astype(F32)
        for hh in range(H):
            c = slice(hh * DH, (hh + 1) * DH)
            kh = lax.dot_general(xs, wk_ref[:, c], _DN(1, 0),
                                 preferred_element_type=F32)
            k_ref[hh, rows, :] = _rope(kh, cos_s, sin_s).astype(BF16)
            v_ref[hh, rows, :] = lax.dot_general(
                xs, wv_ref[:, c], _DN(1, 0),
                preferred_element_type=F32).astype(BF16)

    def ab_step(h, carry):
        o = lax.rem(my - h + N_DEV, N_DEV)
        rows = pl.ds(o * SQ, SQ)
        hm = jnp.minimum(h, N_DEV - 2)

        def _ag_rdma():
            return pltpu.make_async_remote_copy(
                src_ref=xg_ref.at[rows, :],
                dst_ref=xg_ref.at[rows, :],
                send_sem=ag_send.at[hm],
                recv_sem=ag_recv.at[hm],
                device_id=(right,),
                device_id_type=pl.DeviceIdType.MESH,
            )

        @pl.when(h < N_DEV - 1)
        def _():
            _ag_rdma().start()

        kv_chunk(o)

        @pl.when(h < N_DEV - 1)
        def _():
            _ag_rdma().wait()

        return carry

    lax.fori_loop(0, N_DEV, ab_step, 0)

    def rs_step(t, carry):
        c = lax.rem(my - t - 1 + 2 * N_DEV, N_DEV)
        rows = pl.ds(c * SQ, SQ)
        xs = xg_ref[rows, :]
        cos_s = cos_ref[rows, :].astype(F32)
        sin_s = sin_ref[rows, :].astype(F32)
        for hh in range(H):
            cc = slice(hh * DH, (hh + 1) * DH)
            qh = lax.dot_general(xs, wq_ref[:, cc], _DN(1, 0),
                                 preferred_element_type=F32)
            qb_ref[hh, :, :] = (_rope(qh, cos_s, sin_s) * SCALE2).astype(BF16)

        po_ref[...] = jnp.zeros((SQ, D), F32)

        def attn_step(idx, acarry):
            hh = lax.div(idx, NQB)
            qb = lax.rem(idx, NQB)
            q_sub = qb_ref[hh, pl.ds(qb * QB, QB), :]
            s_mat = lax.dot_general(q_sub, k_ref[hh], _DN(1, 1),
                                    preferred_element_type=F32)
            p = jnp.exp2(s_mat.astype(BF16))
            denom = jnp.sum(p, axis=1, keepdims=True, dtype=F32)
            cv = lax.dot_general(p, v_ref[hh], _DN(1, 0),
                                 preferred_element_type=F32) / denom
            contrib = lax.dot_general(cv.astype(BF16),
                                      wo_ref[pl.ds(hh * DH, DH), :],
                                      _DN(1, 0), preferred_element_type=F32)
            rr = pl.ds(qb * QB, QB)
            po_ref[rr, :] = po_ref[rr, :] + contrib
            return acarry

        lax.fori_loop(0, H * NQB, attn_step, 0)

        def _rs_rdma(i):
            return pltpu.make_async_remote_copy(
                src_ref=sb_ref.at[lax.rem(i, 2)],
                dst_ref=rs_ref.at[lax.rem(i, 2)],
                send_sem=rs_send.at[i],
                recv_sem=rs_recv.at[i],
                device_id=(right,),
                device_id_type=pl.DeviceIdType.MESH,
            )

        @pl.when(t > 0)
        def _add_recv():
            tm1 = jnp.maximum(t - 1, 0)
            _rs_rdma(tm1).wait_recv()
            po_ref[...] = po_ref[...] + rs_ref[lax.rem(tm1, 2)].astype(F32)

        @pl.when(t < N_DEV - 1)
        def _send():
            ts = jnp.minimum(t, N_DEV - 2)

            @pl.when(t >= 2)
            def _reclaim():
                _rs_rdma(jnp.maximum(t - 2, 0)).wait_send()

            slot = lax.rem(ts, 2)
            sb_ref[slot] = po_ref[...].astype(BF16)
            _rs_rdma(ts).start()

        @pl.when(t == N_DEV - 1)
        def _emit():
            _rs_rdma(1).wait_send()
            _rs_rdma(2).wait_send()
            sb_ref[1] = po_ref[...].astype(BF16)
            copy = pltpu.make_async_copy(sb_ref.at[1], out_ref, out_sem)
            copy.start()
            copy.wait()

        return carry

    lax.fori_loop(0, N_DEV, rs_step, 0)


def kernel(x, Wq, Wk, Wv, Wo):
    cos_np, sin_np, rot_np = _rope_tables()
    cos = jnp.asarray(cos_np, dtype=BF16)
    sin = jnp.asarray(sin_np, dtype=BF16)
    rot = jnp.asarray(rot_np, dtype=BF16)

    hbm = pltpu.MemorySpace.HBM
    xb = x.reshape(SQ, D).astype(BF16)
    xb = pltpu.with_memory_space_constraint(xb, hbm)
    cos = pltpu.with_memory_space_constraint(cos, hbm)
    sin = pltpu.with_memory_space_constraint(sin, hbm)
    rot = pltpu.with_memory_space_constraint(rot, hbm)
    wq = Wq.astype(BF16)
    wk = Wk.astype(BF16)
    wv = Wv.astype(BF16)
    wo = Wo.astype(BF16)

    out = pl.pallas_call(
        _body,
        out_shape=jax.ShapeDtypeStruct((SQ, D), BF16),
        in_specs=(
            [pl.BlockSpec(memory_space=hbm)]
            + [pl.BlockSpec(memory_space=pltpu.VMEM)] * 4
            + [pl.BlockSpec(memory_space=hbm)] * 3
        ),
        out_specs=pl.BlockSpec(memory_space=pltpu.MemorySpace.HBM),
        input_output_aliases={0: 0},
        scratch_shapes=[
            pltpu.VMEM((S, D), BF16),
            pltpu.VMEM((H, S, DH), BF16),
            pltpu.VMEM((H, S, DH), BF16),
            pltpu.VMEM((H, SQ, DH), BF16),
            pltpu.VMEM((SQ, D), F32),
            pltpu.VMEM((2, SQ, D), BF16),
            pltpu.VMEM((2, SQ, D), BF16),
            pltpu.VMEM((S, DH), BF16),
            pltpu.VMEM((S, DH), BF16),
            pltpu.VMEM((DH, DH), BF16),
            pltpu.SemaphoreType.DMA((N_DEV - 1,)),
            pltpu.SemaphoreType.DMA((N_DEV - 1,)),
            pltpu.SemaphoreType.DMA((N_DEV - 1,)),
            pltpu.SemaphoreType.DMA((N_DEV - 1,)),
            pltpu.SemaphoreType.DMA((4,)),
            pltpu.SemaphoreType.DMA,
        ],
        compiler_params=pltpu.CompilerParams(
            collective_id=0,
            vmem_limit_bytes=51 * 1024 * 1024,
        ),
    )(xb, wq, wk, wv, wo, cos, sin, rot)
    return out.reshape(1, SQ, D)


# device time: 354643 ns/iter; 1.0250x vs baseline; 1.0250x over previous
import jax
import jax.numpy as jnp
import numpy as np
from jax import lax
from jax.experimental import pallas as pl
from jax.experimental.pallas import tpu as pltpu

N_DEV = 4
SQ = 1024
S = N_DEV * SQ
D = 1024
H = 8
DH = 128
QB = 256
NQB = SQ // QB
SCALE = 0.08838834764831843
SCALE2 = SCALE * 1.4426950408889634

F32 = jnp.float32
BF16 = jnp.bfloat16

_DN = lambda lc, rc: (((lc,), (rc,)), ((), ()))


def _rope_tables():
    inv = 1.0 / (10000.0 ** (np.arange(0, DH, 2) / DH))
    pos = np.arange(S)[:, None] * inv[None, :]
    cos = np.repeat(np.cos(pos), 2, axis=-1)
    sin = np.repeat(np.sin(pos), 2, axis=-1)
    R = np.zeros((DH, DH), np.float32)
    for k in range(DH // 2):
        R[2 * k + 1, 2 * k] = -1.0
        R[2 * k, 2 * k + 1] = 1.0
    return cos.astype(np.float32), sin.astype(np.float32), R


def _body(x_ref, wq_ref, wk_ref, wv_ref, wo_ref, cosh_ref, sinh_ref, roth_ref,
          out_ref, xg_ref, k_ref, v_ref, qb_ref, po_ref, sb_ref, rs_ref,
          cos_ref, sin_ref, rot_ref,
          ag_send, ag_recv, rs_send, rs_recv, dma_sems, out_sem):
    my = lax.axis_index("i")
    left = lax.rem(my + N_DEV - 1, N_DEV)
    right = lax.rem(my + 1, N_DEV)

    barrier = pltpu.get_barrier_semaphore()
    for nbr in (left, right):
        pl.semaphore_signal(barrier, inc=1, device_id=(nbr,),
                            device_id_type=pl.DeviceIdType.MESH)
    pl.semaphore_wait(barrier, 2)

    copies = [
        pltpu.make_async_copy(x_ref, xg_ref.at[pl.ds(my * SQ, SQ), :],
                              dma_sems.at[0]),
        pltpu.make_async_copy(cosh_ref, cos_ref, dma_sems.at[1]),
        pltpu.make_async_copy(sinh_ref, sin_ref, dma_sems.at[2]),
        pltpu.make_async_copy(roth_ref, rot_ref, dma_sems.at[3]),
    ]
    for cp in copies:
        cp.start()
    for cp in copies:
        cp.wait()

    rot = rot_ref[...]

    def _rope(tf, cos_s, sin_s):
        tr = lax.dot_general(tf.astype(BF16), rot, _DN(1, 0),
                             preferred_element_type=F32)
        return tf * cos_s + tr * sin_s

    def kv_chunk(o):
        rows = pl.ds(o * SQ, SQ)
        xs = xg_ref[rows, :]
        cos_s = cos_ref[rows, :].astype(F32)
        sin_s = sin_ref[rows, :].astype(F32)
        for hh in range(H):
            c = slice(hh * DH, (hh + 1) * DH)
            kh = lax.dot_general(xs, wk_ref[:, c], _DN(1, 0),
                                 preferred_element_type=F32)
            k_ref[hh, rows, :] = _rope(kh, cos_s, sin_s).astype(BF16)
            v_ref[hh, rows, :] = lax.dot_general(
                xs, wv_ref[:, c], _DN(1, 0),
                preferred_element_type=F32).astype(BF16)

    def ab_step(h, carry):
        o = lax.rem(my - h + N_DEV, N_DEV)
        rows = pl.ds(o * SQ, SQ)
        hm = jnp.minimum(h, N_DEV - 2)

        def _ag_rdma():
            return pltpu.make_async_remote_copy(
                src_ref=xg_ref.at[rows, :],
                dst_ref=xg_ref.at[rows, :],
                send_sem=ag_send.at[hm],
                recv_sem=ag_recv.at[hm],
                device_id=(right,),
                device_id_type=pl.DeviceIdType.MESH,
            )

        @pl.when(h < N_DEV - 1)
        def _():
            _ag_rdma().start()

        kv_chunk(o)

        @pl.when(h < N_DEV - 1)
        def _():
            _ag_rdma().wait()

        return carry

    lax.fori_loop(0, N_DEV, ab_step, 0)

    def rs_step(t, carry):
        c = lax.rem(my - t - 1 + 2 * N_DEV, N_DEV)
        rows = pl.ds(c * SQ, SQ)
        xs = xg_ref[rows, :]
        cos_s = cos_ref[rows, :].astype(F32)
        sin_s = sin_ref[rows, :].astype(F32)
        for hh in range(H):
            cc = slice(hh * DH, (hh + 1) * DH)
            qh = lax.dot_general(xs, wq_ref[:, cc], _DN(1, 0),
                                 preferred_element_type=F32)
            qb_ref[hh, :, :] = (_rope(qh, cos_s, sin_s) * SCALE).astype(BF16)

        po_ref[...] = jnp.zeros((SQ, D), F32)

        def attn_step(idx, acarry):
            hh = lax.div(idx, NQB)
            qb = lax.rem(idx, NQB)
            q_sub = qb_ref[hh, pl.ds(qb * QB, QB), :]
            s_mat = lax.dot_general(q_sub, k_ref[hh], _DN(1, 1),
                                    preferred_element_type=F32)
            p = jnp.exp(s_mat.astype(BF16))
            denom = jnp.sum(p, axis=1, keepdims=True, dtype=F32)
            cv = lax.dot_general(p, v_ref[hh], _DN(1, 0),
                                 preferred_element_type=F32) / denom
            contrib = lax.dot_general(cv.astype(BF16),
                                      wo_ref[pl.ds(hh * DH, DH), :],
                                      _DN(1, 0), preferred_element_type=F32)
            rr = pl.ds(qb * QB, QB)
            po_ref[rr, :] = po_ref[rr, :] + contrib
            return acarry

        lax.fori_loop(0, H * NQB, attn_step, 0)

        def _rs_rdma(i):
            return pltpu.make_async_remote_copy(
                src_ref=sb_ref.at[lax.rem(i, 2)],
                dst_ref=rs_ref.at[lax.rem(i, 2)],
                send_sem=rs_send.at[i],
                recv_sem=rs_recv.at[i],
                device_id=(right,),
                device_id_type=pl.DeviceIdType.MESH,
            )

        @pl.when(t > 0)
        def _add_recv():
            tm1 = jnp.maximum(t - 1, 0)
            _rs_rdma(tm1).wait_recv()
            po_ref[...] = po_ref[...] + rs_ref[lax.rem(tm1, 2)].astype(F32)

        @pl.when(t < N_DEV - 1)
        def _send():
            ts = jnp.minimum(t, N_DEV - 2)

            @pl.when(t >= 2)
            def _reclaim():
                _rs_rdma(jnp.maximum(t - 2, 0)).wait_send()

            slot = lax.rem(ts, 2)
            sb_ref[slot] = po_ref[...].astype(BF16)
            _rs_rdma(ts).start()

        @pl.when(t == N_DEV - 1)
        def _emit():
            _rs_rdma(1).wait_send()
            _rs_rdma(2).wait_send()
            sb_ref[1] = po_ref[...].astype(BF16)
            copy = pltpu.make_async_copy(sb_ref.at[1], out_ref, out_sem)
            copy.start()
            copy.wait()

        return carry

    lax.fori_loop(0, N_DEV, rs_step, 0)


def kernel(x, Wq, Wk, Wv, Wo):
    cos_np, sin_np, rot_np = _rope_tables()
    cos = jnp.asarray(cos_np, dtype=BF16)
    sin = jnp.asarray(sin_np, dtype=BF16)
    rot = jnp.asarray(rot_np, dtype=BF16)

    hbm = pltpu.MemorySpace.HBM
    xb = x.reshape(SQ, D).astype(BF16)
    xb = pltpu.with_memory_space_constraint(xb, hbm)
    cos = pltpu.with_memory_space_constraint(cos, hbm)
    sin = pltpu.with_memory_space_constraint(sin, hbm)
    rot = pltpu.with_memory_space_constraint(rot, hbm)
    wq = Wq.astype(BF16)
    wk = Wk.astype(BF16)
    wv = Wv.astype(BF16)
    wo = Wo.astype(BF16)

    out = pl.pallas_call(
        _body,
        out_shape=jax.ShapeDtypeStruct((SQ, D), BF16),
        in_specs=(
            [pl.BlockSpec(memory_space=hbm)]
            + [pl.BlockSpec(memory_space=pltpu.VMEM)] * 4
            + [pl.BlockSpec(memory_space=hbm)] * 3
        ),
        out_specs=pl.BlockSpec(memory_space=pltpu.MemorySpace.HBM),
        input_output_aliases={0: 0},
        scratch_shapes=[
            pltpu.VMEM((S, D), BF16),
            pltpu.VMEM((H, S, DH), BF16),
            pltpu.VMEM((H, S, DH), BF16),
            pltpu.VMEM((H, SQ, DH), BF16),
            pltpu.VMEM((SQ, D), F32),
            pltpu.VMEM((2, SQ, D), BF16),
            pltpu.VMEM((2, SQ, D), BF16),
            pltpu.VMEM((S, DH), BF16),
            pltpu.VMEM((S, DH), BF16),
            pltpu.VMEM((DH, DH), BF16),
            pltpu.SemaphoreType.DMA((N_DEV - 1,)),
            pltpu.SemaphoreType.DMA((N_DEV - 1,)),
            pltpu.SemaphoreType.DMA((N_DEV - 1,)),
            pltpu.SemaphoreType.DMA((N_DEV - 1,)),
            pltpu.SemaphoreType.DMA((4,)),
            pltpu.SemaphoreType.DMA,
        ],
        compiler_params=pltpu.CompilerParams(
            collective_id=0,
            vmem_limit_bytes=51 * 1024 * 1024,
        ),
    )(xb, wq, wk, wv, wo, cos, sin, rot)
    return out.reshape(1, SQ, D)
